# Initial kernel scaffold; baseline (speedup 1.0000x reference)
import jax
import jax.numpy as jnp
from jax.experimental import pallas as pl


def kernel(h, inputs, edge_index, W1, b1, W2, b2, W_adv, b_adv):
    raise NotImplementedError("write your pallas kernel here")


# speedup vs baseline: 7.1981x; 7.1981x over previous
import functools

import jax
import jax.numpy as jnp
from jax import lax
from jax.experimental import pallas as pl
from jax.experimental.pallas import tpu as pltpu
from jax.experimental.pallas import tpu_sc as plsc

N = 10000
NP = 10240
E = 320000
CW = 80
EROWS = 4096
EP = EROWS * CW
F_IN = 200
F_HID = 128
NC = 2
NS = 16
NODES_PER_TILE = NP // NS
ROWS_PER_WORKER = EROWS // (NC * NS)
ROWS_PER_TILE = EROWS // NS
EDGES_PER_WORKER = ROWS_PER_WORKER * CW
EDGES_PER_TILE = ROWS_PER_TILE * CW

_MESH = plsc.VectorSubcoreMesh(core_axis_name="c", subcore_axis_name="s")


def _zero_fill(ref, n16):
    def body(i, _):
        ref[pl.ds(i * 16, 16)] = jnp.zeros((16,), jnp.float32)
        return 0
    lax.fori_loop(0, n16, body, 0)


def _deg_body(src_hbm, dst_hbm, dego_hbm, degi_hbm,
              idx_v, ones_v, zero_v, acc, s0, s1):
    cid = lax.axis_index("c")
    sid = lax.axis_index("s")

    _zero_fill(zero_v, NODES_PER_TILE // 16)
    pltpu.sync_copy(zero_v, acc.at[pl.ds(sid * NODES_PER_TILE, NODES_PER_TILE)])
    for k in range(CW // 16):
        ones_v[pl.ds(k * 16, 16)] = jnp.ones((16,), jnp.float32)
    plsc.subcore_barrier()

    base = sid * ROWS_PER_TILE

    @pl.when(cid == 0)
    def _():
        pltpu.sync_copy(src_hbm.at[pl.ds(base, ROWS_PER_TILE)], idx_v)

    @pl.when(cid == 1)
    def _():
        pltpu.sync_copy(dst_hbm.at[pl.ds(base, ROWS_PER_TILE)], idx_v)

    def fire(j, sem):
        pltpu.async_copy(ones_v, acc.at[idx_v.at[j]], sem, add=True)

    def drain(sem):
        pltpu.make_async_copy(ones_v, acc.at[idx_v.at[0]], sem).wait()

    fire(0, s0)
    fire(1, s1)

    def body(jj, _):
        drain(s0)
        fire(2 * jj, s0)
        drain(s1)
        fire(2 * jj + 1, s1)
        return 0

    lax.fori_loop(1, ROWS_PER_TILE // 2, body, 0)
    drain(s0)
    drain(s1)
    plsc.subcore_barrier()
    sl = pl.ds(sid * NODES_PER_TILE, NODES_PER_TILE)

    @pl.when(cid == 0)
    def _():
        pltpu.sync_copy(acc.at[sl], dego_hbm.at[sl])

    @pl.when(cid == 1)
    def _():
        pltpu.sync_copy(acc.at[sl], degi_hbm.at[sl])


_sc_deg = pl.kernel(
    _deg_body,
    out_type=[jax.ShapeDtypeStruct((NP,), jnp.float32),
              jax.ShapeDtypeStruct((NP,), jnp.float32)],
    mesh=_MESH,
    scratch_types=[
        pltpu.VMEM((ROWS_PER_TILE, CW), jnp.int32),
        pltpu.VMEM((CW,), jnp.float32),
        pltpu.VMEM((NODES_PER_TILE,), jnp.float32),
        pltpu.VMEM_SHARED((NP,), jnp.float32),
        pltpu.SemaphoreType.DMA,
        pltpu.SemaphoreType.DMA,
    ],
)


def _agg_body(srcf_hbm, dst_hbm, xw_hbm, out_hbm,
              src_v, dst_v, rows0, rows1, acc, s0, s1):
    cid = lax.axis_index("c")
    sid = lax.axis_index("s")
    wid = cid * NS + sid

    def zrow(r, _):
        for k in range(F_HID // 16):
            rows0[r, pl.ds(k * 16, 16)] = jnp.zeros((16,), jnp.float32)
        return 0
    lax.fori_loop(0, CW, zrow, 0)
    for b in range(NODES_PER_TILE // CW):
        pltpu.sync_copy(
            rows0, acc.at[pl.ds(sid * NODES_PER_TILE + b * CW, CW)])
    plsc.subcore_barrier()

    pltpu.sync_copy(
        srcf_hbm.at[pl.ds(wid * EDGES_PER_WORKER, EDGES_PER_WORKER)], src_v)
    pltpu.sync_copy(
        dst_hbm.at[pl.ds(wid * ROWS_PER_WORKER, ROWS_PER_WORKER)], dst_v)

    def fire(j, buf, sem):
        pltpu.async_copy(xw_hbm.at[src_v.at[pl.ds(j * CW, CW)]], buf, sem)

    def drain(buf, sem):
        pltpu.make_async_copy(
            xw_hbm.at[src_v.at[pl.ds(0, CW)]], buf, sem).wait()

    def scat(j, buf):
        pltpu.sync_copy(buf, acc.at[dst_v.at[j]], add=True)

    fire(0, rows0, s0)
    fire(1, rows1, s1)

    def body(jj, _):
        drain(rows0, s0)
        scat(2 * jj, rows0)
        fire(2 * jj + 2, rows0, s0)
        drain(rows1, s1)
        scat(2 * jj + 1, rows1)
        fire(2 * jj + 3, rows1, s1)
        return 0

    lax.fori_loop(0, ROWS_PER_WORKER // 2 - 1, body, 0)
    drain(rows0, s0)
    scat(ROWS_PER_WORKER - 2, rows0)
    drain(rows1, s1)
    scat(ROWS_PER_WORKER - 1, rows1)
    plsc.subcore_barrier()
    sl = pl.ds(sid * NODES_PER_TILE, NODES_PER_TILE)
    pltpu.sync_copy(acc.at[sl], out_hbm.at[cid, sl])


_sc_agg = pl.kernel(
    _agg_body,
    out_type=jax.ShapeDtypeStruct((NC, NP, F_HID), jnp.float32),
    mesh=_MESH,
    scratch_types=[
        pltpu.VMEM((EDGES_PER_WORKER,), jnp.int32),
        pltpu.VMEM((ROWS_PER_WORKER, CW), jnp.int32),
        pltpu.VMEM((CW, F_HID), jnp.float32),
        pltpu.VMEM((CW, F_HID), jnp.float32),
        pltpu.VMEM_SHARED((NP, F_HID), jnp.float32),
        pltpu.SemaphoreType.DMA,
        pltpu.SemaphoreType.DMA,
    ],
)


def _l2_body(srcf_hbm, dst_hbm, z_hbm, nd_hbm, b2_hbm, out_hbm,
             src_v, dst_v, vals0, vals1, q_v, nd_v, s_v, b2_v,
             zero_v, acc, s0, s1):
    cid = lax.axis_index("c")
    sid = lax.axis_index("s")

    @pl.when(cid == 0)
    def _():
        _zero_fill(zero_v, NODES_PER_TILE // 16)
        pltpu.sync_copy(
            zero_v, acc.at[pl.ds(sid * NODES_PER_TILE, NODES_PER_TILE)])
        plsc.subcore_barrier()

        pltpu.sync_copy(
            srcf_hbm.at[pl.ds(sid * EDGES_PER_TILE, EDGES_PER_TILE)], src_v)
        pltpu.sync_copy(
            dst_hbm.at[pl.ds(sid * ROWS_PER_TILE, ROWS_PER_TILE)], dst_v)

        def fire(j, vals, sem):
            pltpu.async_copy(
                z_hbm.at[src_v.at[pl.ds(j * CW, CW)]], vals, sem)

        def drain(vals, sem):
            pltpu.make_async_copy(
                z_hbm.at[src_v.at[pl.ds(0, CW)]], vals, sem).wait()

        def scat(j, vals):
            pltpu.sync_copy(vals, acc.at[dst_v.at[j]], add=True)

        fire(0, vals0, s0)
        fire(1, vals1, s1)

        def body(jj, _):
            drain(vals0, s0)
            scat(2 * jj, vals0)
            fire(2 * jj + 2, vals0, s0)
            drain(vals1, s1)
            scat(2 * jj + 1, vals1)
            fire(2 * jj + 3, vals1, s1)
            return 0

        lax.fori_loop(0, ROWS_PER_TILE // 2 - 1, body, 0)
        drain(vals0, s0)
        scat(ROWS_PER_TILE - 2, vals0)
        drain(vals1, s1)
        scat(ROWS_PER_TILE - 1, vals1)
        plsc.subcore_barrier()

        sl = pl.ds(sid * NODES_PER_TILE, NODES_PER_TILE)
        pltpu.sync_copy(acc.at[sl], q_v)
        pltpu.sync_copy(nd_hbm.at[sl], nd_v)
        pltpu.sync_copy(b2_hbm, b2_v)
        b2vec = b2_v[...]

        def fin(i, _):
            s_v[pl.ds(i * 16, 16)] = (
                q_v[pl.ds(i * 16, 16)] * nd_v[pl.ds(i * 16, 16)] + b2vec)
            return 0

        lax.fori_loop(0, NODES_PER_TILE // 16, fin, 0)
        pltpu.sync_copy(s_v, out_hbm.at[sl])


_sc_l2 = pl.kernel(
    _l2_body,
    out_type=jax.ShapeDtypeStruct((NP,), jnp.float32),
    mesh=_MESH,
    scratch_types=[
        pltpu.VMEM((EDGES_PER_TILE,), jnp.int32),
        pltpu.VMEM((ROWS_PER_TILE, CW), jnp.int32),
        pltpu.VMEM((CW,), jnp.float32),
        pltpu.VMEM((CW,), jnp.float32),
        pltpu.VMEM((NODES_PER_TILE,), jnp.float32),
        pltpu.VMEM((NODES_PER_TILE,), jnp.float32),
        pltpu.VMEM((NODES_PER_TILE,), jnp.float32),
        pltpu.VMEM((16,), jnp.float32),
        pltpu.VMEM((NODES_PER_TILE,), jnp.float32),
        pltpu.VMEM_SHARED((NP,), jnp.float32),
        pltpu.SemaphoreType.DMA,
        pltpu.SemaphoreType.DMA,
    ],
)


_BLK = 512


def _xw_body(x_ref, w_ref, deg_ref, xws_ref, ns_ref, nd_ref):
    deg = deg_ref[...]
    n = jnp.where(deg > 0, lax.rsqrt(jnp.maximum(deg, 1.0)), 0.0)
    ns = n[:, 0:1]
    nd = n[:, 1:2]
    ns_ref[...] = ns
    nd_ref[...] = nd
    xws_ref[...] = jnp.dot(x_ref[...], w_ref[...],
                           preferred_element_type=jnp.float32) * ns


def _tc_xw_scale(x, w1, deg_t):
    return pl.pallas_call(
        _xw_body,
        grid=(NP // _BLK,),
        in_specs=[
            pl.BlockSpec((_BLK, F_IN), lambda i: (i, 0)),
            pl.BlockSpec((F_IN, F_HID), lambda i: (0, 0)),
            pl.BlockSpec((_BLK, 2), lambda i: (i, 0)),
        ],
        out_specs=[
            pl.BlockSpec((_BLK, F_HID), lambda i: (i, 0)),
            pl.BlockSpec((_BLK, 1), lambda i: (i, 0)),
            pl.BlockSpec((_BLK, 1), lambda i: (i, 0)),
        ],
        out_shape=[
            jax.ShapeDtypeStruct((NP, F_HID), jnp.float32),
            jax.ShapeDtypeStruct((NP, 1), jnp.float32),
            jax.ShapeDtypeStruct((NP, 1), jnp.float32),
        ],
    )(x, w1, deg_t)


def _mid_body(p_ref, ns_ref, nd_ref, b1_ref, w2_ref, z_ref):
    agg = p_ref[0] + p_ref[1]
    x1 = jnp.maximum(agg * nd_ref[...] + b1_ref[...], 0.0)
    z_ref[...] = jnp.dot(x1, w2_ref[...],
                         preferred_element_type=jnp.float32) * ns_ref[...]


def _tc_mid(p, ns2, nd2, b1, w2):
    return pl.pallas_call(
        _mid_body,
        grid=(NP // _BLK,),
        in_specs=[
            pl.BlockSpec((NC, _BLK, F_HID), lambda i: (0, i, 0)),
            pl.BlockSpec((_BLK, 1), lambda i: (i, 0)),
            pl.BlockSpec((_BLK, 1), lambda i: (i, 0)),
            pl.BlockSpec((1, F_HID), lambda i: (0, 0)),
            pl.BlockSpec((F_HID, 1), lambda i: (0, 0)),
        ],
        out_specs=pl.BlockSpec((_BLK, 1), lambda i: (i, 0)),
        out_shape=jax.ShapeDtypeStruct((NP, 1), jnp.float32),
    )(p, ns2, nd2, b1, w2)


def _adv_body(h_ref, w_ref, b_ref, o_ref):
    o_ref[...] = jnp.dot(h_ref[...], w_ref[...],
                         preferred_element_type=jnp.float32) + b_ref[...]


def _tc_adv(h, w_adv_t, b_adv):
    blk = 400
    return pl.pallas_call(
        _adv_body,
        grid=(N // blk,),
        in_specs=[
            pl.BlockSpec((blk, 256), lambda i: (i, 0)),
            pl.BlockSpec((256, 1), lambda i: (0, 0)),
            pl.BlockSpec((1, 1), lambda i: (0, 0)),
        ],
        out_specs=pl.BlockSpec((blk, 1), lambda i: (i, 0)),
        out_shape=jax.ShapeDtypeStruct((N, 1), jnp.float32),
    )(h, w_adv_t, b_adv)


@jax.jit
def kernel(h, inputs, edge_index, W1, b1, W2, b2, W_adv, b_adv):
    ei = edge_index.astype(jnp.int32)
    srcf = jnp.pad(ei[0], (0, EP - E), constant_values=NP - 1)
    dstf = jnp.pad(ei[1], (0, EP - E), constant_values=NP - 1)
    src2d = srcf.reshape(EROWS, CW)
    dst2d = dstf.reshape(EROWS, CW)
    x0p = jnp.pad(inputs[0], ((0, NP - N), (0, 0)))

    dego, degi = _sc_deg(src2d, dst2d)
    deg_t = jnp.stack([dego, degi], axis=1)
    xws, ns2, nd2 = _tc_xw_scale(x0p, W1, deg_t)
    p = _sc_agg(srcf, dst2d, xws)
    z2 = _tc_mid(p, ns2, nd2, b1.reshape(1, F_HID), W2)
    sp = _sc_l2(srcf, dst2d, z2.reshape(NP), nd2.reshape(NP),
                jnp.broadcast_to(b2, (16,)))
    s = sp[:N].reshape(N, 1)
    s_g = _tc_adv(h, W_adv.T, b_adv.reshape(1, 1))
    return (s, s_g)


# speedup vs baseline: 8.5340x; 1.1856x over previous
import functools

import jax
import jax.numpy as jnp
from jax import lax
from jax.experimental import pallas as pl
from jax.experimental.pallas import tpu as pltpu
from jax.experimental.pallas import tpu_sc as plsc

N = 10000
NP = 10240
E = 320000
CW = 128
EROWS = 2560
EP = EROWS * CW
F_IN = 200
F_HID = 128
NC = 2
NS = 16
NODES_PER_TILE = NP // NS
ROWS_PER_WORKER = EROWS // (NC * NS)
ROWS_PER_TILE = EROWS // NS
EDGES_PER_WORKER = ROWS_PER_WORKER * CW
EDGES_PER_TILE = ROWS_PER_TILE * CW

_MESH = plsc.VectorSubcoreMesh(core_axis_name="c", subcore_axis_name="s")


def _zero_fill(ref, n16):
    def body(i, _):
        ref[pl.ds(i * 16, 16)] = jnp.zeros((16,), jnp.float32)
        return 0
    lax.fori_loop(0, n16, body, 0)


def _deg_body(src_hbm, dst_hbm, dego_hbm, degi_hbm,
              idx_v, ones_v, zero_v, acc, s0, s1):
    cid = lax.axis_index("c")
    sid = lax.axis_index("s")

    _zero_fill(zero_v, NODES_PER_TILE // 16)
    pltpu.sync_copy(zero_v, acc.at[pl.ds(sid * NODES_PER_TILE, NODES_PER_TILE)])
    for k in range(CW // 16):
        ones_v[pl.ds(k * 16, 16)] = jnp.ones((16,), jnp.float32)
    plsc.subcore_barrier()

    base = sid * ROWS_PER_TILE

    @pl.when(cid == 0)
    def _():
        pltpu.sync_copy(src_hbm.at[pl.ds(base, ROWS_PER_TILE)], idx_v)

    @pl.when(cid == 1)
    def _():
        pltpu.sync_copy(dst_hbm.at[pl.ds(base, ROWS_PER_TILE)], idx_v)

    def fire(j, sem):
        pltpu.async_copy(ones_v, acc.at[idx_v.at[j]], sem, add=True)

    def drain(sem):
        pltpu.make_async_copy(ones_v, acc.at[idx_v.at[0]], sem).wait()

    fire(0, s0)
    fire(1, s1)

    def body(jj, _):
        drain(s0)
        fire(2 * jj, s0)
        drain(s1)
        fire(2 * jj + 1, s1)
        return 0

    lax.fori_loop(1, ROWS_PER_TILE // 2, body, 0)
    drain(s0)
    drain(s1)
    plsc.subcore_barrier()
    sl = pl.ds(sid * NODES_PER_TILE, NODES_PER_TILE)

    @pl.when(cid == 0)
    def _():
        pltpu.sync_copy(acc.at[sl], dego_hbm.at[sl])

    @pl.when(cid == 1)
    def _():
        pltpu.sync_copy(acc.at[sl], degi_hbm.at[sl])


_sc_deg = pl.kernel(
    _deg_body,
    out_type=[jax.ShapeDtypeStruct((NP,), jnp.float32),
              jax.ShapeDtypeStruct((NP,), jnp.float32)],
    mesh=_MESH,
    scratch_types=[
        pltpu.VMEM((ROWS_PER_TILE, CW), jnp.int32),
        pltpu.VMEM((CW,), jnp.float32),
        pltpu.VMEM((NODES_PER_TILE,), jnp.float32),
        pltpu.VMEM_SHARED((NP,), jnp.float32),
        pltpu.SemaphoreType.DMA,
        pltpu.SemaphoreType.DMA,
    ],
)


def _agg_body(srcf_hbm, dst_hbm, xw_hbm, out_hbm,
              src_v, dst_v, rows0, rows1, acc, s0, s1):
    cid = lax.axis_index("c")
    sid = lax.axis_index("s")
    wid = cid * NS + sid

    def zrow(r, _):
        for k in range(F_HID // 16):
            rows0[r, pl.ds(k * 16, 16)] = jnp.zeros((16,), jnp.float32)
        return 0
    lax.fori_loop(0, CW, zrow, 0)
    for b in range(NODES_PER_TILE // CW):
        pltpu.sync_copy(
            rows0, acc.at[pl.ds(sid * NODES_PER_TILE + b * CW, CW)])
    plsc.subcore_barrier()

    pltpu.sync_copy(
        srcf_hbm.at[pl.ds(wid * EDGES_PER_WORKER, EDGES_PER_WORKER)], src_v)

    HALF = ROWS_PER_WORKER // 2

    def fire(j, buf, sem):
        pltpu.async_copy(xw_hbm.at[src_v.at[pl.ds(j * CW, CW)]], buf, sem)

    def drain(buf, sem):
        pltpu.make_async_copy(
            xw_hbm.at[src_v.at[pl.ds(0, CW)]], buf, sem).wait()

    def scat(jl, buf):
        pltpu.sync_copy(buf, acc.at[dst_v.at[jl]], add=True)

    fire(0, rows0, s0)
    fire(1, rows1, s1)

    for ph in range(2):
        pltpu.sync_copy(
            dst_hbm.at[pl.ds(wid * ROWS_PER_WORKER + ph * HALF, HALF)], dst_v)
        npairs = HALF // 2 - ph

        def body(jj, _, ph=ph):
            j0 = ph * HALF + 2 * jj
            drain(rows0, s0)
            scat(2 * jj, rows0)
            fire(j0 + 2, rows0, s0)
            drain(rows1, s1)
            scat(2 * jj + 1, rows1)
            fire(j0 + 3, rows1, s1)
            return 0

        lax.fori_loop(0, npairs, body, 0)

    drain(rows0, s0)
    scat(HALF - 2, rows0)
    drain(rows1, s1)
    scat(HALF - 1, rows1)
    plsc.subcore_barrier()
    sl = pl.ds(sid * NODES_PER_TILE, NODES_PER_TILE)
    pltpu.sync_copy(acc.at[sl], out_hbm.at[cid, sl])


_sc_agg = pl.kernel(
    _agg_body,
    out_type=jax.ShapeDtypeStruct((NC, NP, F_HID), jnp.float32),
    mesh=_MESH,
    scratch_types=[
        pltpu.VMEM((EDGES_PER_WORKER,), jnp.int32),
        pltpu.VMEM((ROWS_PER_WORKER // 2, CW), jnp.int32),
        pltpu.VMEM((CW, F_HID), jnp.float32),
        pltpu.VMEM((CW, F_HID), jnp.float32),
        pltpu.VMEM_SHARED((NP, F_HID), jnp.float32),
        pltpu.SemaphoreType.DMA,
        pltpu.SemaphoreType.DMA,
    ],
)


def _l2_body(srcf_hbm, dst_hbm, z_hbm, nd_hbm, b2_hbm, out_hbm,
             src_v, dst_v, vals0, vals1, q_v, nd_v, s_v, b2_v,
             zero_v, acc, s0, s1):
    cid = lax.axis_index("c")
    sid = lax.axis_index("s")

    @pl.when(cid == 0)
    def _():
        _zero_fill(zero_v, NODES_PER_TILE // 16)
        pltpu.sync_copy(
            zero_v, acc.at[pl.ds(sid * NODES_PER_TILE, NODES_PER_TILE)])
        plsc.subcore_barrier()

        pltpu.sync_copy(
            srcf_hbm.at[pl.ds(sid * EDGES_PER_TILE, EDGES_PER_TILE)], src_v)
        pltpu.sync_copy(
            dst_hbm.at[pl.ds(sid * ROWS_PER_TILE, ROWS_PER_TILE)], dst_v)

        def fire(j, vals, sem):
            pltpu.async_copy(
                z_hbm.at[src_v.at[pl.ds(j * CW, CW)]], vals, sem)

        def drain(vals, sem):
            pltpu.make_async_copy(
                z_hbm.at[src_v.at[pl.ds(0, CW)]], vals, sem).wait()

        def scat(j, vals):
            pltpu.sync_copy(vals, acc.at[dst_v.at[j]], add=True)

        fire(0, vals0, s0)
        fire(1, vals1, s1)

        def body(jj, _):
            drain(vals0, s0)
            scat(2 * jj, vals0)
            fire(2 * jj + 2, vals0, s0)
            drain(vals1, s1)
            scat(2 * jj + 1, vals1)
            fire(2 * jj + 3, vals1, s1)
            return 0

        lax.fori_loop(0, ROWS_PER_TILE // 2 - 1, body, 0)
        drain(vals0, s0)
        scat(ROWS_PER_TILE - 2, vals0)
        drain(vals1, s1)
        scat(ROWS_PER_TILE - 1, vals1)
        plsc.subcore_barrier()

        sl = pl.ds(sid * NODES_PER_TILE, NODES_PER_TILE)
        pltpu.sync_copy(acc.at[sl], q_v)
        pltpu.sync_copy(nd_hbm.at[sl], nd_v)
        pltpu.sync_copy(b2_hbm, b2_v)
        b2vec = b2_v[...]

        def fin(i, _):
            s_v[pl.ds(i * 16, 16)] = (
                q_v[pl.ds(i * 16, 16)] * nd_v[pl.ds(i * 16, 16)] + b2vec)
            return 0

        lax.fori_loop(0, NODES_PER_TILE // 16, fin, 0)
        pltpu.sync_copy(s_v, out_hbm.at[sl])


_sc_l2 = pl.kernel(
    _l2_body,
    out_type=jax.ShapeDtypeStruct((NP,), jnp.float32),
    mesh=_MESH,
    scratch_types=[
        pltpu.VMEM((EDGES_PER_TILE,), jnp.int32),
        pltpu.VMEM((ROWS_PER_TILE, CW), jnp.int32),
        pltpu.VMEM((CW,), jnp.float32),
        pltpu.VMEM((CW,), jnp.float32),
        pltpu.VMEM((NODES_PER_TILE,), jnp.float32),
        pltpu.VMEM((NODES_PER_TILE,), jnp.float32),
        pltpu.VMEM((NODES_PER_TILE,), jnp.float32),
        pltpu.VMEM((16,), jnp.float32),
        pltpu.VMEM((NODES_PER_TILE,), jnp.float32),
        pltpu.VMEM_SHARED((NP,), jnp.float32),
        pltpu.SemaphoreType.DMA,
        pltpu.SemaphoreType.DMA,
    ],
)


_BLK = 512


def _xw_body(x_ref, w_ref, deg_ref, xws_ref, ns_ref, nd_ref):
    deg = deg_ref[...]
    n = jnp.where(deg > 0, lax.rsqrt(jnp.maximum(deg, 1.0)), 0.0)
    ns = n[:, 0:1]
    nd = n[:, 1:2]
    ns_ref[...] = ns
    nd_ref[...] = nd
    xws_ref[...] = jnp.dot(x_ref[...], w_ref[...],
                           preferred_element_type=jnp.float32) * ns


def _tc_xw_scale(x, w1, deg_t):
    return pl.pallas_call(
        _xw_body,
        grid=(NP // _BLK,),
        in_specs=[
            pl.BlockSpec((_BLK, F_IN), lambda i: (i, 0)),
            pl.BlockSpec((F_IN, F_HID), lambda i: (0, 0)),
            pl.BlockSpec((_BLK, 2), lambda i: (i, 0)),
        ],
        out_specs=[
            pl.BlockSpec((_BLK, F_HID), lambda i: (i, 0)),
            pl.BlockSpec((_BLK, 1), lambda i: (i, 0)),
            pl.BlockSpec((_BLK, 1), lambda i: (i, 0)),
        ],
        out_shape=[
            jax.ShapeDtypeStruct((NP, F_HID), jnp.float32),
            jax.ShapeDtypeStruct((NP, 1), jnp.float32),
            jax.ShapeDtypeStruct((NP, 1), jnp.float32),
        ],
    )(x, w1, deg_t)


def _mid_body(p_ref, ns_ref, nd_ref, b1_ref, w2_ref, z_ref):
    agg = p_ref[0] + p_ref[1]
    x1 = jnp.maximum(agg * nd_ref[...] + b1_ref[...], 0.0)
    z_ref[...] = jnp.dot(x1, w2_ref[...],
                         preferred_element_type=jnp.float32) * ns_ref[...]


def _tc_mid(p, ns2, nd2, b1, w2):
    return pl.pallas_call(
        _mid_body,
        grid=(NP // _BLK,),
        in_specs=[
            pl.BlockSpec((NC, _BLK, F_HID), lambda i: (0, i, 0)),
            pl.BlockSpec((_BLK, 1), lambda i: (i, 0)),
            pl.BlockSpec((_BLK, 1), lambda i: (i, 0)),
            pl.BlockSpec((1, F_HID), lambda i: (0, 0)),
            pl.BlockSpec((F_HID, 1), lambda i: (0, 0)),
        ],
        out_specs=pl.BlockSpec((_BLK, 1), lambda i: (i, 0)),
        out_shape=jax.ShapeDtypeStruct((NP, 1), jnp.float32),
    )(p, ns2, nd2, b1, w2)


def _adv_body(h_ref, w_ref, b_ref, o_ref):
    o_ref[...] = jnp.dot(h_ref[...], w_ref[...],
                         preferred_element_type=jnp.float32) + b_ref[...]


def _tc_adv(h, w_adv_t, b_adv):
    blk = 400
    return pl.pallas_call(
        _adv_body,
        grid=(N // blk,),
        in_specs=[
            pl.BlockSpec((blk, 256), lambda i: (i, 0)),
            pl.BlockSpec((256, 1), lambda i: (0, 0)),
            pl.BlockSpec((1, 1), lambda i: (0, 0)),
        ],
        out_specs=pl.BlockSpec((blk, 1), lambda i: (i, 0)),
        out_shape=jax.ShapeDtypeStruct((N, 1), jnp.float32),
    )(h, w_adv_t, b_adv)


@jax.jit
def kernel(h, inputs, edge_index, W1, b1, W2, b2, W_adv, b_adv):
    ei = edge_index.astype(jnp.int32)
    srcf = jnp.pad(ei[0], (0, EP - E), constant_values=NP - 1)
    dstf = jnp.pad(ei[1], (0, EP - E), constant_values=NP - 1)
    src2d = srcf.reshape(EROWS, CW)
    dst2d = dstf.reshape(EROWS, CW)
    x0p = jnp.pad(inputs[0], ((0, NP - N), (0, 0)))

    dego, degi = _sc_deg(src2d, dst2d)
    deg_t = jnp.stack([dego, degi], axis=1)
    xws, ns2, nd2 = _tc_xw_scale(x0p, W1, deg_t)
    p = _sc_agg(srcf, dst2d, xws)
    z2 = _tc_mid(p, ns2, nd2, b1.reshape(1, F_HID), W2)
    sp = _sc_l2(srcf, dst2d, z2.reshape(NP), nd2.reshape(NP),
                jnp.broadcast_to(b2, (16,)))
    s = sp[:N].reshape(N, 1)
    s_g = _tc_adv(h, W_adv.T, b_adv.reshape(1, 1))
    return (s, s_g)


# speedup vs baseline: 8.6164x; 1.0097x over previous
import functools

import jax
import jax.numpy as jnp
from jax import lax
from jax.experimental import pallas as pl
from jax.experimental.pallas import tpu as pltpu
from jax.experimental.pallas import tpu_sc as plsc

N = 10000
NP = 10240
E = 320000
CW = 128
EROWS = 2560
EP = EROWS * CW
F_IN = 200
F_HID = 128
NC = 2
NS = 16
NODES_PER_TILE = NP // NS
ROWS_PER_WORKER = EROWS // (NC * NS)
ROWS_PER_TILE = EROWS // NS
EDGES_PER_WORKER = ROWS_PER_WORKER * CW
EDGES_PER_TILE = ROWS_PER_TILE * CW

_MESH = plsc.VectorSubcoreMesh(core_axis_name="c", subcore_axis_name="s")


def _zero_fill(ref, n16):
    def body(i, _):
        ref[pl.ds(i * 16, 16)] = jnp.zeros((16,), jnp.float32)
        return 0
    lax.fori_loop(0, n16, body, 0)


def _deg_body(src_hbm, dst_hbm, dego_hbm, degi_hbm,
              idx_v, ones_v, zero_v, acc, s0, s1):
    cid = lax.axis_index("c")
    sid = lax.axis_index("s")

    _zero_fill(zero_v, NODES_PER_TILE // 16)
    pltpu.sync_copy(zero_v, acc.at[pl.ds(sid * NODES_PER_TILE, NODES_PER_TILE)])
    for k in range(CW // 16):
        ones_v[pl.ds(k * 16, 16)] = jnp.ones((16,), jnp.float32)
    plsc.subcore_barrier()

    base = sid * ROWS_PER_TILE

    @pl.when(cid == 0)
    def _():
        pltpu.sync_copy(src_hbm.at[pl.ds(base, ROWS_PER_TILE)], idx_v)

    @pl.when(cid == 1)
    def _():
        pltpu.sync_copy(dst_hbm.at[pl.ds(base, ROWS_PER_TILE)], idx_v)

    def fire(j, sem):
        pltpu.async_copy(ones_v, acc.at[idx_v.at[j]], sem, add=True)

    def drain(sem):
        pltpu.make_async_copy(ones_v, acc.at[idx_v.at[0]], sem).wait()

    fire(0, s0)
    fire(1, s1)

    def body(jj, _):
        drain(s0)
        fire(2 * jj, s0)
        drain(s1)
        fire(2 * jj + 1, s1)
        return 0

    lax.fori_loop(1, ROWS_PER_TILE // 2, body, 0)
    drain(s0)
    drain(s1)
    plsc.subcore_barrier()
    sl = pl.ds(sid * NODES_PER_TILE, NODES_PER_TILE)

    @pl.when(cid == 0)
    def _():
        pltpu.sync_copy(acc.at[sl], dego_hbm.at[sl])

    @pl.when(cid == 1)
    def _():
        pltpu.sync_copy(acc.at[sl], degi_hbm.at[sl])


_sc_deg = pl.kernel(
    _deg_body,
    out_type=[jax.ShapeDtypeStruct((NP,), jnp.float32),
              jax.ShapeDtypeStruct((NP,), jnp.float32)],
    mesh=_MESH,
    scratch_types=[
        pltpu.VMEM((ROWS_PER_TILE, CW), jnp.int32),
        pltpu.VMEM((CW,), jnp.float32),
        pltpu.VMEM((NODES_PER_TILE,), jnp.float32),
        pltpu.VMEM_SHARED((NP,), jnp.float32),
        pltpu.SemaphoreType.DMA,
        pltpu.SemaphoreType.DMA,
    ],
)


AGG_C0 = 120
AGG_C1 = 40
assert NS * (AGG_C0 + AGG_C1) == EROWS


def _agg_body(srcf_hbm, dstf_hbm, xw_hbm, out_hbm,
              sidx0, sidx1, didx0, didx1, rows0, rows1, acc,
              gs0, gs1, is0, is1, ds0, ds1):
    cid = lax.axis_index("c")
    sid = lax.axis_index("s")

    def zrow(r, _):
        for k in range(F_HID // 16):
            rows0[r, pl.ds(k * 16, 16)] = jnp.zeros((16,), jnp.float32)
        return 0
    lax.fori_loop(0, CW, zrow, 0)
    for b in range(NODES_PER_TILE // CW):
        pltpu.sync_copy(
            rows0, acc.at[pl.ds(sid * NODES_PER_TILE + b * CW, CW)])
    plsc.subcore_barrier()

    nch = jnp.where(cid == 0, AGG_C0, AGG_C1)
    base = jnp.where(cid == 0, sid * AGG_C0, NS * AGG_C0 + sid * AGG_C1)

    def fire_i(j, sbuf, dbuf, ssem, dsem):
        pltpu.async_copy(srcf_hbm.at[pl.ds(j * CW, CW)], sbuf, ssem)
        pltpu.async_copy(dstf_hbm.at[pl.ds(j * CW, CW)], dbuf, dsem)

    def wait_i(buf, sem):
        pltpu.make_async_copy(srcf_hbm.at[pl.ds(0, CW)], buf, sem).wait()

    def fire_g(sbuf, buf, sem):
        pltpu.async_copy(xw_hbm.at[sbuf], buf, sem)

    def wait_g(buf, sem):
        pltpu.make_async_copy(xw_hbm.at[sidx0], buf, sem).wait()

    def scat(dbuf, buf):
        pltpu.sync_copy(buf, acc.at[dbuf], add=True)

    fire_i(base, sidx0, didx0, is0, ds0)
    fire_i(base + 1, sidx1, didx1, is1, ds1)
    wait_i(sidx0, is0)
    fire_g(sidx0, rows0, gs0)
    wait_i(sidx1, is1)
    fire_g(sidx1, rows1, gs1)

    def body(jj, _):
        a = base + 2 * jj
        wait_g(rows0, gs0)
        pltpu.async_copy(srcf_hbm.at[pl.ds((a + 2) * CW, CW)], sidx0, is0)
        wait_i(didx0, ds0)
        scat(didx0, rows0)
        pltpu.async_copy(dstf_hbm.at[pl.ds((a + 2) * CW, CW)], didx0, ds0)
        wait_i(sidx0, is0)
        fire_g(sidx0, rows0, gs0)

        wait_g(rows1, gs1)
        pltpu.async_copy(srcf_hbm.at[pl.ds((a + 3) * CW, CW)], sidx1, is1)
        wait_i(didx1, ds1)
        scat(didx1, rows1)
        pltpu.async_copy(dstf_hbm.at[pl.ds((a + 3) * CW, CW)], didx1, ds1)
        wait_i(sidx1, is1)
        fire_g(sidx1, rows1, gs1)
        return 0

    lax.fori_loop(0, nch // 2 - 1, body, 0)
    wait_g(rows0, gs0)
    wait_i(didx0, ds0)
    scat(didx0, rows0)
    wait_g(rows1, gs1)
    wait_i(didx1, ds1)
    scat(didx1, rows1)
    plsc.subcore_barrier()
    sl = pl.ds(sid * NODES_PER_TILE, NODES_PER_TILE)
    pltpu.sync_copy(acc.at[sl], out_hbm.at[cid, sl])


_sc_agg = pl.kernel(
    _agg_body,
    out_type=jax.ShapeDtypeStruct((NC, NP, F_HID), jnp.float32),
    mesh=_MESH,
    scratch_types=[
        pltpu.VMEM((CW,), jnp.int32),
        pltpu.VMEM((CW,), jnp.int32),
        pltpu.VMEM((CW,), jnp.int32),
        pltpu.VMEM((CW,), jnp.int32),
        pltpu.VMEM((CW, F_HID), jnp.float32),
        pltpu.VMEM((CW, F_HID), jnp.float32),
        pltpu.VMEM_SHARED((NP, F_HID), jnp.float32),
        pltpu.SemaphoreType.DMA,
        pltpu.SemaphoreType.DMA,
        pltpu.SemaphoreType.DMA,
        pltpu.SemaphoreType.DMA,
        pltpu.SemaphoreType.DMA,
        pltpu.SemaphoreType.DMA,
    ],
)


def _l2_body(srcf_hbm, dst_hbm, z_hbm, nd_hbm, b2_hbm, out_hbm,
             src_v, dst_v, vals0, vals1, q_v, nd_v, s_v, b2_v,
             zero_v, acc, s0, s1):
    cid = lax.axis_index("c")
    sid = lax.axis_index("s")

    @pl.when(cid == 0)
    def _():
        _zero_fill(zero_v, NODES_PER_TILE // 16)
        pltpu.sync_copy(
            zero_v, acc.at[pl.ds(sid * NODES_PER_TILE, NODES_PER_TILE)])
        plsc.subcore_barrier()

        pltpu.sync_copy(
            srcf_hbm.at[pl.ds(sid * EDGES_PER_TILE, EDGES_PER_TILE)], src_v)
        pltpu.sync_copy(
            dst_hbm.at[pl.ds(sid * ROWS_PER_TILE, ROWS_PER_TILE)], dst_v)

        def fire(j, vals, sem):
            pltpu.async_copy(
                z_hbm.at[src_v.at[pl.ds(j * CW, CW)]], vals, sem)

        def drain(vals, sem):
            pltpu.make_async_copy(
                z_hbm.at[src_v.at[pl.ds(0, CW)]], vals, sem).wait()

        def scat(j, vals):
            pltpu.sync_copy(vals, acc.at[dst_v.at[j]], add=True)

        fire(0, vals0, s0)
        fire(1, vals1, s1)

        def body(jj, _):
            drain(vals0, s0)
            scat(2 * jj, vals0)
            fire(2 * jj + 2, vals0, s0)
            drain(vals1, s1)
            scat(2 * jj + 1, vals1)
            fire(2 * jj + 3, vals1, s1)
            return 0

        lax.fori_loop(0, ROWS_PER_TILE // 2 - 1, body, 0)
        drain(vals0, s0)
        scat(ROWS_PER_TILE - 2, vals0)
        drain(vals1, s1)
        scat(ROWS_PER_TILE - 1, vals1)
        plsc.subcore_barrier()

        sl = pl.ds(sid * NODES_PER_TILE, NODES_PER_TILE)
        pltpu.sync_copy(acc.at[sl], q_v)
        pltpu.sync_copy(nd_hbm.at[sl], nd_v)
        pltpu.sync_copy(b2_hbm, b2_v)
        b2vec = b2_v[...]

        def fin(i, _):
            s_v[pl.ds(i * 16, 16)] = (
                q_v[pl.ds(i * 16, 16)] * nd_v[pl.ds(i * 16, 16)] + b2vec)
            return 0

        lax.fori_loop(0, NODES_PER_TILE // 16, fin, 0)
        pltpu.sync_copy(s_v, out_hbm.at[sl])


_sc_l2 = pl.kernel(
    _l2_body,
    out_type=jax.ShapeDtypeStruct((NP,), jnp.float32),
    mesh=_MESH,
    scratch_types=[
        pltpu.VMEM((EDGES_PER_TILE,), jnp.int32),
        pltpu.VMEM((ROWS_PER_TILE, CW), jnp.int32),
        pltpu.VMEM((CW,), jnp.float32),
        pltpu.VMEM((CW,), jnp.float32),
        pltpu.VMEM((NODES_PER_TILE,), jnp.float32),
        pltpu.VMEM((NODES_PER_TILE,), jnp.float32),
        pltpu.VMEM((NODES_PER_TILE,), jnp.float32),
        pltpu.VMEM((16,), jnp.float32),
        pltpu.VMEM((NODES_PER_TILE,), jnp.float32),
        pltpu.VMEM_SHARED((NP,), jnp.float32),
        pltpu.SemaphoreType.DMA,
        pltpu.SemaphoreType.DMA,
    ],
)


_BLK = 512


def _xw_body(x_ref, w_ref, deg_ref, xws_ref, ns_ref, nd_ref):
    deg = deg_ref[...]
    n = jnp.where(deg > 0, lax.rsqrt(jnp.maximum(deg, 1.0)), 0.0)
    ns = n[:, 0:1]
    nd = n[:, 1:2]
    ns_ref[...] = ns
    nd_ref[...] = nd
    xws_ref[...] = jnp.dot(x_ref[...], w_ref[...],
                           preferred_element_type=jnp.float32) * ns


def _tc_xw_scale(x, w1, deg_t):
    return pl.pallas_call(
        _xw_body,
        grid=(NP // _BLK,),
        in_specs=[
            pl.BlockSpec((_BLK, F_IN), lambda i: (i, 0)),
            pl.BlockSpec((F_IN, F_HID), lambda i: (0, 0)),
            pl.BlockSpec((_BLK, 2), lambda i: (i, 0)),
        ],
        out_specs=[
            pl.BlockSpec((_BLK, F_HID), lambda i: (i, 0)),
            pl.BlockSpec((_BLK, 1), lambda i: (i, 0)),
            pl.BlockSpec((_BLK, 1), lambda i: (i, 0)),
        ],
        out_shape=[
            jax.ShapeDtypeStruct((NP, F_HID), jnp.float32),
            jax.ShapeDtypeStruct((NP, 1), jnp.float32),
            jax.ShapeDtypeStruct((NP, 1), jnp.float32),
        ],
    )(x, w1, deg_t)


def _mid_body(p_ref, ns_ref, nd_ref, b1_ref, w2_ref, z_ref):
    agg = p_ref[0] + p_ref[1]
    x1 = jnp.maximum(agg * nd_ref[...] + b1_ref[...], 0.0)
    z_ref[...] = jnp.dot(x1, w2_ref[...],
                         preferred_element_type=jnp.float32) * ns_ref[...]


def _tc_mid(p, ns2, nd2, b1, w2):
    return pl.pallas_call(
        _mid_body,
        grid=(NP // _BLK,),
        in_specs=[
            pl.BlockSpec((NC, _BLK, F_HID), lambda i: (0, i, 0)),
            pl.BlockSpec((_BLK, 1), lambda i: (i, 0)),
            pl.BlockSpec((_BLK, 1), lambda i: (i, 0)),
            pl.BlockSpec((1, F_HID), lambda i: (0, 0)),
            pl.BlockSpec((F_HID, 1), lambda i: (0, 0)),
        ],
        out_specs=pl.BlockSpec((_BLK, 1), lambda i: (i, 0)),
        out_shape=jax.ShapeDtypeStruct((NP, 1), jnp.float32),
    )(p, ns2, nd2, b1, w2)


def _adv_body(h_ref, w_ref, b_ref, o_ref):
    o_ref[...] = jnp.dot(h_ref[...], w_ref[...],
                         preferred_element_type=jnp.float32) + b_ref[...]


def _tc_adv(h, w_adv_t, b_adv):
    blk = 400
    return pl.pallas_call(
        _adv_body,
        grid=(N // blk,),
        in_specs=[
            pl.BlockSpec((blk, 256), lambda i: (i, 0)),
            pl.BlockSpec((256, 1), lambda i: (0, 0)),
            pl.BlockSpec((1, 1), lambda i: (0, 0)),
        ],
        out_specs=pl.BlockSpec((blk, 1), lambda i: (i, 0)),
        out_shape=jax.ShapeDtypeStruct((N, 1), jnp.float32),
    )(h, w_adv_t, b_adv)


@jax.jit
def kernel(h, inputs, edge_index, W1, b1, W2, b2, W_adv, b_adv):
    ei = edge_index.astype(jnp.int32)
    srcf = jnp.pad(ei[0], (0, EP - E), constant_values=NP - 1)
    dstf = jnp.pad(ei[1], (0, EP - E), constant_values=NP - 1)
    src2d = srcf.reshape(EROWS, CW)
    dst2d = dstf.reshape(EROWS, CW)
    x0p = jnp.pad(inputs[0], ((0, NP - N), (0, 0)))

    dego, degi = _sc_deg(src2d, dst2d)
    deg_t = jnp.stack([dego, degi], axis=1)
    xws, ns2, nd2 = _tc_xw_scale(x0p, W1, deg_t)
    p = _sc_agg(srcf, dstf, xws)
    z2 = _tc_mid(p, ns2, nd2, b1.reshape(1, F_HID), W2)
    sp = _sc_l2(srcf, dst2d, z2.reshape(NP), nd2.reshape(NP),
                jnp.broadcast_to(b2, (16,)))
    s = sp[:N].reshape(N, 1)
    s_g = _tc_adv(h, W_adv.T, b_adv.reshape(1, 1))
    return (s, s_g)
